# Initial kernel scaffold; baseline (speedup 1.0000x reference)
import jax
import jax.numpy as jnp
from jax.experimental import pallas as pl


def kernel(x, edge_index, W1, b1, W2, b2, W3, b3):
    raise NotImplementedError("write your pallas kernel here")


# speedup vs baseline: 7.0105x; 7.0105x over previous
import functools

import jax
import jax.numpy as jnp
from jax import lax
from jax.experimental import pallas as pl
from jax.experimental.pallas import tpu as pltpu
from jax.experimental.pallas import tpu_sc as plsc

N = 10000
E = 320000
D = 128

NC = 2
NS = 16
NW = NC * NS
CH = 128
EPT = 10240
EPAD = EPT * NW
CPT = EPT // CH
NPAD = NW * 316
RPT = NPAD // NS

_mesh = plsc.VectorSubcoreMesh(core_axis_name="c", subcore_axis_name="s")


@functools.partial(
    pl.kernel,
    out_type=jax.ShapeDtypeStruct((NW, NPAD), jnp.float32),
    mesh=_mesh,
    scratch_types=[
        pltpu.VMEM((EPT,), jnp.int32),
        pltpu.VMEM((NPAD,), jnp.float32),
    ],
    compiler_params=pltpu.CompilerParams(needs_layout_passes=False),
)
def _sc_deg(col_hbm, out_hbm, colbuf, deg):
    cid = lax.axis_index("c")
    sid = lax.axis_index("s")
    wid = cid * NS + sid

    def zero(i, carry):
        deg[pl.ds(i * 16, 16)] = jnp.zeros((16,), jnp.float32)
        return carry

    lax.fori_loop(0, NPAD // 16, zero, 0)

    pltpu.sync_copy(col_hbm.at[pl.ds(wid * EPT, EPT)], colbuf)
    ones = jnp.ones((16,), jnp.float32)

    def count(i, carry):
        idx = colbuf[pl.ds(i * 16, 16)]
        plsc.addupdate_scatter(deg, [idx], ones)
        return carry

    lax.fori_loop(0, EPT // 16, count, 0)
    pltpu.sync_copy(deg, out_hbm.at[wid])


@functools.partial(
    pl.kernel,
    out_type=jax.ShapeDtypeStruct((NC, NPAD, D), jnp.float32),
    mesh=_mesh,
    scratch_types=[
        pltpu.VMEM((EPT,), jnp.int32),
        pltpu.VMEM((CPT, CH), jnp.int32),
        pltpu.VMEM((CH, D), jnp.float32),
        pltpu.VMEM_SHARED((NPAD, D), jnp.float32),
        pltpu.SemaphoreType.DMA,
    ],
)
def _sc_scatter(xw_hbm, row_hbm, col2d_hbm, zeros_hbm, out_hbm,
                rowbuf, colbuf, msg, acc, sem):
    cid = lax.axis_index("c")
    sid = lax.axis_index("s")
    wid = cid * NS + sid

    pltpu.sync_copy(zeros_hbm, acc.at[pl.ds(sid * RPT, RPT)])
    pltpu.sync_copy(row_hbm.at[pl.ds(wid * EPT, EPT)], rowbuf)
    pltpu.sync_copy(col2d_hbm.at[pl.ds(wid * CPT, CPT)], colbuf)
    plsc.subcore_barrier()

    def body(c, carry):
        idx = rowbuf.at[pl.ds(c * CH, CH)]
        pltpu.async_copy(xw_hbm.at[idx], msg, sem).wait()
        pltpu.sync_copy(msg, acc.at[colbuf.at[c]], add=True)
        return carry

    lax.fori_loop(0, CPT, body, 0)
    plsc.subcore_barrier()
    pltpu.sync_copy(acc.at[pl.ds(sid * RPT, RPT)],
                    out_hbm.at[cid, pl.ds(sid * RPT, RPT)])


BLK = 1000


def _tc_dinv_body(degp_ref, dinv_ref):
    deg = 1.0 + jnp.sum(degp_ref[...], axis=0)
    dinv_ref[...] = lax.rsqrt(deg).reshape(NPAD, 1)


def _tc_pre_body(x_ref, w_ref, dinv_ref, xw_ref):
    xw_ref[...] = dinv_ref[...] * jnp.dot(x_ref[...], w_ref[...],
                                          preferred_element_type=jnp.float32,
                                          precision=lax.Precision.HIGHEST)


def _tc_mid_body(acc_ref, xwp_ref, dinv_ref, b_ref, w_ref, out_ref):
    dinv = dinv_ref[...]
    a = acc_ref[0] + acc_ref[1] + xwp_ref[...]
    h = jnp.maximum(dinv * a + b_ref[...], 0.0)
    out_ref[...] = dinv * jnp.dot(h, w_ref[...],
                                  preferred_element_type=jnp.float32,
                                  precision=lax.Precision.HIGHEST)


def _tc_post_body(acc_ref, xwp_ref, dinv_ref, b_ref, out_ref):
    a = acc_ref[0] + acc_ref[1] + xwp_ref[...]
    out_ref[...] = dinv_ref[...] * a + b_ref[...]


_spec_rows = pl.BlockSpec((BLK, D), lambda i: (i, 0))
_spec_w = pl.BlockSpec((D, D), lambda i: (0, 0))
_spec_b = pl.BlockSpec((1, D), lambda i: (0, 0))
_spec_dinv = pl.BlockSpec((BLK, 1), lambda i: (i, 0))
_spec_acc = pl.BlockSpec((NC, BLK, D), lambda i: (0, i, 0))

_tc_dinv = pl.pallas_call(
    _tc_dinv_body,
    out_shape=jax.ShapeDtypeStruct((NPAD, 1), jnp.float32),
)

_tc_pre = pl.pallas_call(
    _tc_pre_body,
    grid=(N // BLK,),
    in_specs=[_spec_rows, _spec_w, _spec_dinv],
    out_specs=_spec_rows,
    out_shape=jax.ShapeDtypeStruct((N, D), jnp.float32),
)

_tc_mid = pl.pallas_call(
    _tc_mid_body,
    grid=(N // BLK,),
    in_specs=[_spec_acc, _spec_rows, _spec_dinv, _spec_b, _spec_w],
    out_specs=_spec_rows,
    out_shape=jax.ShapeDtypeStruct((N, D), jnp.float32),
)

_tc_post = pl.pallas_call(
    _tc_post_body,
    grid=(N // BLK,),
    in_specs=[_spec_acc, _spec_rows, _spec_dinv, _spec_b],
    out_specs=_spec_rows,
    out_shape=jax.ShapeDtypeStruct((N, D), jnp.float32),
)


@jax.jit
def kernel(x, edge_index, W1, b1, W2, b2, W3, b3):
    row = edge_index[0].astype(jnp.int32)
    col = edge_index[1].astype(jnp.int32)
    rowp = jnp.concatenate([row, jnp.zeros((EPAD - E,), jnp.int32)])
    colp = jnp.concatenate([col, jnp.full((EPAD - E,), N, jnp.int32)])
    col2d = colp.reshape(EPAD // CH, CH)
    zeros = jnp.zeros((RPT, D), jnp.float32)

    degp = _sc_deg(colp)
    dinv = _tc_dinv(degp)
    xw1 = _tc_pre(x, W1, dinv)
    acc = _sc_scatter(xw1, rowp, col2d, zeros)
    xw2 = _tc_mid(acc, xw1, dinv, b1.reshape(1, D), W2)
    acc = _sc_scatter(xw2, rowp, col2d, zeros)
    xw3 = _tc_mid(acc, xw2, dinv, b2.reshape(1, D), W3)
    acc = _sc_scatter(xw3, rowp, col2d, zeros)
    return _tc_post(acc, xw3, dinv, b3.reshape(1, D))


# speedup vs baseline: 9.2832x; 1.3242x over previous
import functools

import jax
import jax.numpy as jnp
from jax import lax
from jax.experimental import pallas as pl
from jax.experimental.pallas import tpu as pltpu
from jax.experimental.pallas import tpu_sc as plsc

N = 10000
E = 320000
D = 128

NC = 2
NS = 16
NW = NC * NS
CH = 64
EPT = 10240
EPAD = EPT * NW
CPT = EPT // CH
NPAD = NW * 316
RPT = NPAD // NS

_mesh = plsc.VectorSubcoreMesh(core_axis_name="c", subcore_axis_name="s")


@functools.partial(
    pl.kernel,
    out_type=jax.ShapeDtypeStruct((NW, NPAD), jnp.float32),
    mesh=_mesh,
    scratch_types=[
        pltpu.VMEM((EPT,), jnp.int32),
        pltpu.VMEM((NPAD,), jnp.float32),
    ],
    compiler_params=pltpu.CompilerParams(needs_layout_passes=False),
)
def _sc_deg(col_hbm, out_hbm, colbuf, deg):
    cid = lax.axis_index("c")
    sid = lax.axis_index("s")
    wid = cid * NS + sid

    def zero(i, carry):
        deg[pl.ds(i * 16, 16)] = jnp.zeros((16,), jnp.float32)
        return carry

    lax.fori_loop(0, NPAD // 16, zero, 0)

    pltpu.sync_copy(col_hbm.at[pl.ds(wid * EPT, EPT)], colbuf)
    ones = jnp.ones((16,), jnp.float32)

    def count(i, carry):
        idx = colbuf[pl.ds(i * 16, 16)]
        plsc.addupdate_scatter(deg, [idx], ones)
        return carry

    lax.fori_loop(0, EPT // 16, count, 0)
    pltpu.sync_copy(deg, out_hbm.at[wid])


@functools.partial(
    pl.kernel,
    out_type=jax.ShapeDtypeStruct((NC, NPAD, D), jnp.float32),
    mesh=_mesh,
    scratch_types=[
        pltpu.VMEM((EPT,), jnp.int32),
        pltpu.VMEM((CPT, CH), jnp.int32),
        pltpu.VMEM((CH, D), jnp.float32),
        pltpu.VMEM((CH, D), jnp.float32),
        pltpu.VMEM_SHARED((NPAD, D), jnp.float32),
        pltpu.SemaphoreType.DMA,
        pltpu.SemaphoreType.DMA,
    ],
)
def _sc_scatter(xw_hbm, row_hbm, col2d_hbm, zeros_hbm, out_hbm,
                rowbuf, colbuf, msg0, msg1, acc, sem0, sem1):
    cid = lax.axis_index("c")
    sid = lax.axis_index("s")
    wid = cid * NS + sid

    pltpu.sync_copy(zeros_hbm, acc.at[pl.ds(sid * RPT, RPT)])
    pltpu.sync_copy(row_hbm.at[pl.ds(wid * EPT, EPT)], rowbuf)
    pltpu.sync_copy(col2d_hbm.at[pl.ds(wid * CPT, CPT)], colbuf)
    plsc.subcore_barrier()

    def gather(c, msg, sem):
        pltpu.async_copy(xw_hbm.at[rowbuf.at[pl.ds(c * CH, CH)]], msg, sem)

    def wait_gather(c, msg, sem):
        pltpu.make_async_copy(xw_hbm.at[rowbuf.at[pl.ds(c * CH, CH)]],
                              msg, sem).wait()

    def scatter(c, msg):
        pltpu.sync_copy(msg, acc.at[colbuf.at[c]], add=True)

    gather(0, msg0, sem0)
    gather(1, msg1, sem1)

    def body(i, carry):
        c = 2 * i
        wait_gather(c, msg0, sem0)
        scatter(c, msg0)
        gather(c + 2, msg0, sem0)
        wait_gather(c + 1, msg1, sem1)
        scatter(c + 1, msg1)
        gather(c + 3, msg1, sem1)
        return carry

    lax.fori_loop(0, CPT // 2 - 1, body, 0)
    wait_gather(CPT - 2, msg0, sem0)
    scatter(CPT - 2, msg0)
    wait_gather(CPT - 1, msg1, sem1)
    scatter(CPT - 1, msg1)
    plsc.subcore_barrier()
    pltpu.sync_copy(acc.at[pl.ds(sid * RPT, RPT)],
                    out_hbm.at[cid, pl.ds(sid * RPT, RPT)])


BLK = 1000


def _tc_dinv_body(degp_ref, dinv_ref):
    deg = 1.0 + jnp.sum(degp_ref[...], axis=0)
    dinv_ref[...] = lax.rsqrt(deg).reshape(NPAD, 1)


def _tc_pre_body(x_ref, w_ref, dinv_ref, xw_ref):
    xw_ref[...] = dinv_ref[...] * jnp.dot(x_ref[...], w_ref[...],
                                          preferred_element_type=jnp.float32,
                                          precision=lax.Precision.HIGHEST)


def _tc_mid_body(acc_ref, xwp_ref, dinv_ref, b_ref, w_ref, out_ref):
    dinv = dinv_ref[...]
    a = acc_ref[0] + acc_ref[1] + xwp_ref[...]
    h = jnp.maximum(dinv * a + b_ref[...], 0.0)
    out_ref[...] = dinv * jnp.dot(h, w_ref[...],
                                  preferred_element_type=jnp.float32,
                                  precision=lax.Precision.HIGHEST)


def _tc_post_body(acc_ref, xwp_ref, dinv_ref, b_ref, out_ref):
    a = acc_ref[0] + acc_ref[1] + xwp_ref[...]
    out_ref[...] = dinv_ref[...] * a + b_ref[...]


_spec_rows = pl.BlockSpec((BLK, D), lambda i: (i, 0))
_spec_w = pl.BlockSpec((D, D), lambda i: (0, 0))
_spec_b = pl.BlockSpec((1, D), lambda i: (0, 0))
_spec_dinv = pl.BlockSpec((BLK, 1), lambda i: (i, 0))
_spec_acc = pl.BlockSpec((NC, BLK, D), lambda i: (0, i, 0))

_tc_dinv = pl.pallas_call(
    _tc_dinv_body,
    out_shape=jax.ShapeDtypeStruct((NPAD, 1), jnp.float32),
)

_tc_pre = pl.pallas_call(
    _tc_pre_body,
    grid=(N // BLK,),
    in_specs=[_spec_rows, _spec_w, _spec_dinv],
    out_specs=_spec_rows,
    out_shape=jax.ShapeDtypeStruct((N, D), jnp.float32),
)

_tc_mid = pl.pallas_call(
    _tc_mid_body,
    grid=(N // BLK,),
    in_specs=[_spec_acc, _spec_rows, _spec_dinv, _spec_b, _spec_w],
    out_specs=_spec_rows,
    out_shape=jax.ShapeDtypeStruct((N, D), jnp.float32),
)

_tc_post = pl.pallas_call(
    _tc_post_body,
    grid=(N // BLK,),
    in_specs=[_spec_acc, _spec_rows, _spec_dinv, _spec_b],
    out_specs=_spec_rows,
    out_shape=jax.ShapeDtypeStruct((N, D), jnp.float32),
)


@jax.jit
def kernel(x, edge_index, W1, b1, W2, b2, W3, b3):
    row = edge_index[0].astype(jnp.int32)
    col = edge_index[1].astype(jnp.int32)
    rowp = jnp.concatenate([row, jnp.zeros((EPAD - E,), jnp.int32)])
    colp = jnp.concatenate([col, jnp.full((EPAD - E,), N, jnp.int32)])
    col2d = colp.reshape(EPAD // CH, CH)
    zeros = jnp.zeros((RPT, D), jnp.float32)

    degp = _sc_deg(colp)
    dinv = _tc_dinv(degp)
    xw1 = _tc_pre(x, W1, dinv)
    acc = _sc_scatter(xw1, rowp, col2d, zeros)
    xw2 = _tc_mid(acc, xw1, dinv, b1.reshape(1, D), W2)
    acc = _sc_scatter(xw2, rowp, col2d, zeros)
    xw3 = _tc_mid(acc, xw2, dinv, b2.reshape(1, D), W3)
    acc = _sc_scatter(xw3, rowp, col2d, zeros)
    return _tc_post(acc, xw3, dinv, b3.reshape(1, D))


# speedup vs baseline: 9.4857x; 1.0218x over previous
import functools

import jax
import jax.numpy as jnp
from jax import lax
from jax.experimental import pallas as pl
from jax.experimental.pallas import tpu as pltpu
from jax.experimental.pallas import tpu_sc as plsc

N = 10000
E = 320000
D = 128

NC = 2
NS = 16
NW = NC * NS
CH = 64
EPT = 10240
EPAD = EPT * NW
CPT = EPT // CH
NPAD = NW * 316
RPT = NPAD // NS

_mesh = plsc.VectorSubcoreMesh(core_axis_name="c", subcore_axis_name="s")


@functools.partial(
    pl.kernel,
    out_type=jax.ShapeDtypeStruct((NW, NPAD), jnp.float32),
    mesh=_mesh,
    scratch_types=[
        pltpu.VMEM((EPT,), jnp.int32),
        pltpu.VMEM((NPAD,), jnp.float32),
    ],
    compiler_params=pltpu.CompilerParams(needs_layout_passes=False),
)
def _sc_deg(col_hbm, out_hbm, colbuf, deg):
    cid = lax.axis_index("c")
    sid = lax.axis_index("s")
    wid = cid * NS + sid

    def zero(i, carry):
        deg[pl.ds(i * 16, 16)] = jnp.zeros((16,), jnp.float32)
        return carry

    lax.fori_loop(0, NPAD // 16, zero, 0)

    pltpu.sync_copy(col_hbm.at[pl.ds(wid * EPT, EPT)], colbuf)
    ones = jnp.ones((16,), jnp.float32)

    def count(i, carry):
        idx = colbuf[pl.ds(i * 16, 16)]
        plsc.addupdate_scatter(deg, [idx], ones)
        return carry

    lax.fori_loop(0, EPT // 16, count, 0)
    pltpu.sync_copy(deg, out_hbm.at[wid])


@functools.partial(
    pl.kernel,
    out_type=jax.ShapeDtypeStruct((NC, NPAD, D), jnp.float32),
    mesh=_mesh,
    scratch_types=[
        pltpu.VMEM((EPT // 2,), jnp.int32),
        pltpu.VMEM((CPT // 2, CH), jnp.int32),
        pltpu.VMEM((CH, D), jnp.float32),
        pltpu.VMEM((CH, D), jnp.float32),
        pltpu.VMEM((CH, D), jnp.float32),
        pltpu.VMEM((CH, D), jnp.float32),
        pltpu.VMEM_SHARED((NPAD, D), jnp.float32),
        pltpu.SemaphoreType.DMA,
        pltpu.SemaphoreType.DMA,
        pltpu.SemaphoreType.DMA,
        pltpu.SemaphoreType.DMA,
    ],
)
def _sc_scatter(xw_hbm, row_hbm, col2d_hbm, zeros_hbm, out_hbm,
                rowbuf, colbuf, m0, m1, m2, m3, acc, s0, s1, s2, s3):
    cid = lax.axis_index("c")
    sid = lax.axis_index("s")
    wid = cid * NS + sid
    bufs = ((m0, s0), (m1, s1), (m2, s2), (m3, s3))
    NB = len(bufs)
    EPP = EPT // 2
    CPP = CPT // 2

    pltpu.sync_copy(zeros_hbm, acc.at[pl.ds(sid * RPT, RPT)])
    plsc.subcore_barrier()

    def gather(c, msg, sem):
        pltpu.async_copy(xw_hbm.at[rowbuf.at[pl.ds(c * CH, CH)]], msg, sem)

    def wait_gather(c, msg, sem):
        pltpu.make_async_copy(xw_hbm.at[rowbuf.at[pl.ds(c * CH, CH)]],
                              msg, sem).wait()

    def scatter(c, msg):
        pltpu.sync_copy(msg, acc.at[colbuf.at[c]], add=True)

    for h in (0, 1):
        base = wid * EPT + h * EPP
        cbase = wid * CPT + h * CPP
        pltpu.sync_copy(row_hbm.at[pl.ds(base, EPP)], rowbuf)
        pltpu.sync_copy(col2d_hbm.at[pl.ds(cbase, CPP)], colbuf)
        for b in range(NB):
            gather(b, *bufs[b])

        def body(i, carry):
            c = NB * i
            for b in range(NB):
                wait_gather(c + b, *bufs[b])
                scatter(c + b, bufs[b][0])
                gather(c + b + NB, *bufs[b])
            return carry

        lax.fori_loop(0, CPP // NB - 1, body, 0)
        ctail = CPP - NB
        for b in range(NB):
            wait_gather(ctail + b, *bufs[b])
            scatter(ctail + b, bufs[b][0])

    plsc.subcore_barrier()
    pltpu.sync_copy(acc.at[pl.ds(sid * RPT, RPT)],
                    out_hbm.at[cid, pl.ds(sid * RPT, RPT)])


BLK = 1000


def _tc_dinv_body(degp_ref, dinv_ref):
    deg = 1.0 + jnp.sum(degp_ref[...], axis=0)
    dinv_ref[...] = lax.rsqrt(deg).reshape(NPAD, 1)


def _tc_pre_body(x_ref, w_ref, dinv_ref, xw_ref):
    xw_ref[...] = dinv_ref[...] * jnp.dot(x_ref[...], w_ref[...],
                                          preferred_element_type=jnp.float32,
                                          precision=lax.Precision.HIGHEST)


def _tc_mid_body(acc_ref, xwp_ref, dinv_ref, b_ref, w_ref, out_ref):
    dinv = dinv_ref[...]
    a = acc_ref[0] + acc_ref[1] + xwp_ref[...]
    h = jnp.maximum(dinv * a + b_ref[...], 0.0)
    out_ref[...] = dinv * jnp.dot(h, w_ref[...],
                                  preferred_element_type=jnp.float32,
                                  precision=lax.Precision.HIGHEST)


def _tc_post_body(acc_ref, xwp_ref, dinv_ref, b_ref, out_ref):
    a = acc_ref[0] + acc_ref[1] + xwp_ref[...]
    out_ref[...] = dinv_ref[...] * a + b_ref[...]


_spec_rows = pl.BlockSpec((BLK, D), lambda i: (i, 0))
_spec_w = pl.BlockSpec((D, D), lambda i: (0, 0))
_spec_b = pl.BlockSpec((1, D), lambda i: (0, 0))
_spec_dinv = pl.BlockSpec((BLK, 1), lambda i: (i, 0))
_spec_acc = pl.BlockSpec((NC, BLK, D), lambda i: (0, i, 0))

_tc_dinv = pl.pallas_call(
    _tc_dinv_body,
    out_shape=jax.ShapeDtypeStruct((NPAD, 1), jnp.float32),
)

_tc_pre = pl.pallas_call(
    _tc_pre_body,
    grid=(N // BLK,),
    in_specs=[_spec_rows, _spec_w, _spec_dinv],
    out_specs=_spec_rows,
    out_shape=jax.ShapeDtypeStruct((N, D), jnp.float32),
)

_tc_mid = pl.pallas_call(
    _tc_mid_body,
    grid=(N // BLK,),
    in_specs=[_spec_acc, _spec_rows, _spec_dinv, _spec_b, _spec_w],
    out_specs=_spec_rows,
    out_shape=jax.ShapeDtypeStruct((N, D), jnp.float32),
)

_tc_post = pl.pallas_call(
    _tc_post_body,
    grid=(N // BLK,),
    in_specs=[_spec_acc, _spec_rows, _spec_dinv, _spec_b],
    out_specs=_spec_rows,
    out_shape=jax.ShapeDtypeStruct((N, D), jnp.float32),
)


@jax.jit
def kernel(x, edge_index, W1, b1, W2, b2, W3, b3):
    row = edge_index[0].astype(jnp.int32)
    col = edge_index[1].astype(jnp.int32)
    rowp = jnp.concatenate([row, jnp.zeros((EPAD - E,), jnp.int32)])
    colp = jnp.concatenate([col, jnp.full((EPAD - E,), N, jnp.int32)])
    col2d = colp.reshape(EPAD // CH, CH)
    zeros = jnp.zeros((RPT, D), jnp.float32)

    degp = _sc_deg(colp)
    dinv = _tc_dinv(degp)
    xw1 = _tc_pre(x, W1, dinv)
    acc = _sc_scatter(xw1, rowp, col2d, zeros)
    xw2 = _tc_mid(acc, xw1, dinv, b1.reshape(1, D), W2)
    acc = _sc_scatter(xw2, rowp, col2d, zeros)
    xw3 = _tc_mid(acc, xw2, dinv, b2.reshape(1, D), W3)
    acc = _sc_scatter(xw3, rowp, col2d, zeros)
    return _tc_post(acc, xw3, dinv, b3.reshape(1, D))
